# baseline (speedup 1.0000x reference)
import jax
import jax.numpy as jnp
from jax.experimental import pallas as pl
from jax.experimental.pallas import tpu as pltpu


def _copy_all_kernel(pdd_in, pvd_in, pdr_in, pvr_in, cell_in,
                     pdd_out, pvd_out, pdr_out, pvr_out, cell_out,
                     *sems):
    copies = [
        pltpu.make_async_copy(pdd_in, pdd_out, sems[0]),
        pltpu.make_async_copy(pvd_in, pvd_out, sems[1]),
        pltpu.make_async_copy(pdr_in, pdr_out, sems[2]),
        pltpu.make_async_copy(pvr_in, pvr_out, sems[3]),
        pltpu.make_async_copy(cell_in, cell_out, sems[4]),
    ]
    for c in copies:
        c.start()
    for c in copies:
        c.wait()


def kernel(natoms, pred_distance_displace, pred_var_displace,
           pred_distance_relaxed, pred_var_relaxed, pred_cell):
    any_spec = pl.BlockSpec(memory_space=pl.ANY)
    outs = pl.pallas_call(
        _copy_all_kernel,
        in_specs=[any_spec] * 5,
        out_specs=[any_spec] * 5,
        out_shape=[
            jax.ShapeDtypeStruct(pred_distance_displace.shape, jnp.float32),
            jax.ShapeDtypeStruct(pred_var_displace.shape, jnp.float32),
            jax.ShapeDtypeStruct(pred_distance_relaxed.shape, jnp.float32),
            jax.ShapeDtypeStruct(pred_var_relaxed.shape, jnp.float32),
            jax.ShapeDtypeStruct(pred_cell.shape, jnp.float32),
        ],
        scratch_shapes=[pltpu.SemaphoreType.DMA] * 5,
    )(pred_distance_displace, pred_var_displace,
      pred_distance_relaxed, pred_var_relaxed, pred_cell)
    return tuple(outs)
